# baseline (speedup 1.0000x reference)
import functools

import jax
import jax.numpy as jnp
from jax import lax
from jax.experimental import pallas as pl
from jax.experimental.pallas import tpu as pltpu
from jax.experimental.pallas import tpu_sc as plsc

LEAK = 0.1
EOUT = 32
NOUT = 32
EB = 1000

_INTERPRET = False


def _lk(v):
    return jnp.where(v >= 0, v, LEAK * v)


def _xbn_body(x_ref, g_ref, b_ref, xbn_ref):
    x = x_ref[...]
    m = jnp.mean(x, axis=0, keepdims=True)
    v = jnp.mean((x - m) * (x - m), axis=0, keepdims=True)
    xbn_ref[...] = (x - m) * lax.rsqrt(v + 1e-5) * g_ref[...] + b_ref[...]


def _xbn_call(x, g, b):
    N, D = x.shape
    return pl.pallas_call(
        _xbn_body,
        out_shape=jax.ShapeDtypeStruct((N, D), jnp.float32),
        interpret=_INTERPRET,
    )(x, g.reshape(1, D), b.reshape(1, D))


def _estats_body(e_ref, est_ref):
    @pl.when(pl.program_id(0) == 0)
    def _():
        est_ref[...] = jnp.zeros_like(est_ref)

    ch = e_ref[...]
    s = jnp.sum(ch, axis=0, keepdims=True)
    ss = jnp.sum(ch * ch, axis=0, keepdims=True)
    est_ref[...] += jnp.concatenate([s, ss], axis=0)


def _estats_call(e):
    E, ein = e.shape
    CH = 10000
    return pl.pallas_call(
        _estats_body,
        grid=(E // CH,),
        in_specs=[pl.BlockSpec((CH, ein), lambda g: (g, 0))],
        out_specs=pl.BlockSpec((2, ein), lambda g: (0, 0)),
        out_shape=jax.ShapeDtypeStruct((2, ein), jnp.float32),
        interpret=_INTERPRET,
    )(e)


_NC, _NS = 2, 16
_NW = _NC * _NS
_CHK = 128


def _gather_body(EPW, D, table_ref, row_ref, col_ref, src_ref, tgt_ref,
                 idx_a, buf_a, idx_b, buf_b, sem_a, sem_b):
    wid = lax.axis_index("s") * _NC + lax.axis_index("c")
    base = wid * EPW
    nch = (EPW + _CHK - 1) // _CHK

    def body(k, carry):
        off = base + jnp.minimum(k * _CHK, EPW - _CHK)
        pltpu.sync_copy(row_ref.at[pl.ds(off, _CHK)], idx_a)
        pltpu.sync_copy(col_ref.at[pl.ds(off, _CHK)], idx_b)
        ca = pltpu.async_copy(table_ref.at[idx_a], buf_a, sem_a)
        cb = pltpu.async_copy(table_ref.at[idx_b], buf_b, sem_b)
        ca.wait()
        pltpu.sync_copy(buf_a, src_ref.at[pl.ds(off, _CHK)])
        cb.wait()
        pltpu.sync_copy(buf_b, tgt_ref.at[pl.ds(off, _CHK)])
        return carry

    lax.fori_loop(0, nch, body, 0)


def _gather2_call(table, row, col):
    N, D = table.shape
    E = row.shape[0]
    EPW = E // _NW
    mesh = plsc.VectorSubcoreMesh(core_axis_name="c", subcore_axis_name="s")
    kfn = pl.kernel(
        functools.partial(_gather_body, EPW, D),
        out_type=(
            jax.ShapeDtypeStruct((E, D), jnp.float32),
            jax.ShapeDtypeStruct((E, D), jnp.float32),
        ),
        mesh=mesh,
        compiler_params=pltpu.CompilerParams(use_tc_tiling_on_sc=False),
        scratch_types=[
            pltpu.VMEM((_CHK,), jnp.int32),
            pltpu.VMEM((_CHK, D), jnp.float32),
            pltpu.VMEM((_CHK,), jnp.int32),
            pltpu.VMEM((_CHK, D), jnp.float32),
            pltpu.SemaphoreType.DMA,
            pltpu.SemaphoreType.DMA,
        ],
    )
    return kfn(table, row, col)


def _scatter_body(N, NFULL, msg_ref, col_ref, zeros_ref, agg_ref,
                  idx_v, buf_v, shared):
    cid = lax.axis_index("c")
    sid = lax.axis_index("s")
    wid = sid * _NC + cid

    @pl.when(sid == 0)
    def _():
        pltpu.sync_copy(zeros_ref, shared)

    plsc.subcore_barrier()

    def one_chunk(off):
        pltpu.sync_copy(col_ref.at[pl.ds(off, _CHK)], idx_v)
        pltpu.sync_copy(msg_ref.at[pl.ds(off, _CHK)], buf_v)
        pltpu.sync_copy(buf_v, shared.at[idx_v], add=True)

    per_w = NFULL // _NW
    base = wid * per_w * _CHK

    def body(k, carry):
        one_chunk(base + k * _CHK)
        return carry

    lax.fori_loop(0, per_w, body, 0)
    tail = NFULL - per_w * _NW

    @pl.when(wid < tail)
    def _():
        one_chunk((per_w * _NW + wid) * _CHK)

    plsc.subcore_barrier()
    rows = N // _NS
    pltpu.sync_copy(shared.at[pl.ds(sid * rows, rows)],
                    agg_ref.at[cid, pl.ds(sid * rows, rows)])


def _scatter_call(msg, col, N):
    E, D = msg.shape
    nfull = E // _CHK
    zeros = jnp.zeros((N, D), jnp.float32)
    mesh = plsc.VectorSubcoreMesh(core_axis_name="c", subcore_axis_name="s")
    kfn = pl.kernel(
        functools.partial(_scatter_body, N, nfull),
        out_type=jax.ShapeDtypeStruct((_NC, N, D), jnp.float32),
        mesh=mesh,
        compiler_params=pltpu.CompilerParams(use_tc_tiling_on_sc=False),
        scratch_types=[
            pltpu.VMEM((_CHK,), jnp.int32),
            pltpu.VMEM((_CHK, D), jnp.float32),
            pltpu.VMEM_SHARED((N, D), jnp.float32),
        ],
    )
    return kfn(msg, col, zeros)


def _edge_body(nin, ein, has_aff, *refs):
    if has_aff:
        (src_ref, tgt_ref, e_ref, a_ref, c_ref, c1_ref, c2_ref, beeb_ref,
         w1_ref, b1_ref, w2_ref, b2_ref, enew_ref, msg_ref) = refs
    else:
        (src_ref, tgt_ref, e_ref, c1_ref, c2_ref, beeb_ref,
         w1_ref, b1_ref, w2_ref, b2_ref, enew_ref, msg_ref) = refs
    s = src_ref[...]
    t = tgt_ref[...]
    ef = e_ref[...]
    if has_aff:
        ef = ef * a_ref[...] + c_ref[...]
    a1 = jnp.dot(t, c1_ref[...], preferred_element_type=jnp.float32)
    z = functools.reduce(
        jnp.add,
        [s[:, i:i + 1] * a1[:, i * EOUT:(i + 1) * EOUT] for i in range(nin)])
    a2 = jnp.dot(z, c2_ref[...], preferred_element_type=jnp.float32)
    z2 = functools.reduce(
        jnp.add,
        [ef[:, j:j + 1] * a2[:, j * EOUT:(j + 1) * EOUT] for j in range(ein)])
    en = _lk(z2 + beeb_ref[...])
    enew_ref[...] = en
    h1 = _lk(jnp.dot(en, w1_ref[...], preferred_element_type=jnp.float32)
             + b1_ref[...])
    h2 = _lk(jnp.dot(h1, w2_ref[...], preferred_element_type=jnp.float32)
             + b2_ref[...])
    msg_ref[...] = functools.reduce(
        jnp.add,
        [s[:, i:i + 1] * h2[:, i * NOUT:(i + 1) * NOUT] for i in range(nin)])


def _edge_call(nin, ein, src, tgt, ef, aff, L):
    E = src.shape[0]
    c1 = L['bst_w'].transpose(2, 1, 0).reshape(nin, nin * EOUT)
    c2 = L['bee_w'].transpose(1, 2, 0).reshape(EOUT, ein * EOUT)
    beeb = L['bee_b'].reshape(1, EOUT)
    w1 = L['nn1_w'].T
    b1 = L['nn1_b'].reshape(1, nin)
    w2 = L['nn2_w'].T
    b2 = L['nn2_b'].reshape(1, nin * NOUT)
    grid = (E // EB,)
    eb_spec = lambda d: pl.BlockSpec((EB, d), lambda g: (g, 0))
    w_spec = lambda a: pl.BlockSpec(a.shape, lambda g: (0,) * a.ndim)
    ops = [src, tgt, ef]
    specs = [eb_spec(nin), eb_spec(nin), eb_spec(ein)]
    if aff is not None:
        ops += [aff[0], aff[1]]
        specs += [w_spec(aff[0]), w_spec(aff[1])]
    ops += [c1, c2, beeb, w1, b1, w2, b2]
    specs += [w_spec(o) for o in (c1, c2, beeb, w1, b1, w2, b2)]
    return pl.pallas_call(
        functools.partial(_edge_body, nin, ein, aff is not None),
        grid=grid,
        in_specs=specs,
        out_specs=(eb_spec(EOUT), eb_spec(NOUT)),
        out_shape=(
            jax.ShapeDtypeStruct((E, EOUT), jnp.float32),
            jax.ShapeDtypeStruct((E, NOUT), jnp.float32),
        ),
        interpret=_INTERPRET,
    )(*ops)


def _update_body(agg_ref, x_ref, rw_ref, cb_ref, out_ref):
    out_ref[...] = (agg_ref[0] + agg_ref[1]
                    + jnp.dot(x_ref[...], rw_ref[...],
                              preferred_element_type=jnp.float32)
                    + cb_ref[...])


def _update_call(agg2, x, rw, cb):
    N = x.shape[0]
    return pl.pallas_call(
        _update_body,
        out_shape=jax.ShapeDtypeStruct((N, NOUT), jnp.float32),
        interpret=_INTERPRET,
    )(agg2, x, rw.T, cb.reshape(1, NOUT))


def _pred_body(s_ref, t_ref, e_ref, w0s_ref, w0t_ref, w0e_ref, b0_ref,
               w1_ref, b1_ref, w2_ref, b2_ref, w3_ref, b3_ref,
               w4_ref, b4_ref, out_ref):
    dot = lambda a, b: jnp.dot(a, b, preferred_element_type=jnp.float32)
    h = (dot(s_ref[...], w0s_ref[...]) + dot(t_ref[...], w0t_ref[...])
         + dot(e_ref[...], w0e_ref[...]) + b0_ref[...])
    h = _lk(h)
    h = _lk(dot(h, w1_ref[...]) + b1_ref[...])
    h = _lk(dot(h, w2_ref[...]) + b2_ref[...])
    h = _lk(dot(h, w3_ref[...]) + b3_ref[...])
    out_ref[...] = dot(h, w4_ref[...]) + b4_ref[...]


def _pred_call(src, tgt, ef, P):
    E = src.shape[0]
    w0 = P['w0']
    ops = [src, tgt, ef,
           w0[:, :32].T, w0[:, 32:64].T, w0[:, 64:96].T, P['b0'].reshape(1, -1),
           P['w1'].T, P['b1'].reshape(1, -1),
           P['w2'].T, P['b2'].reshape(1, -1),
           P['w3'].T, P['b3'].reshape(1, -1),
           P['w4'].T, P['b4'].reshape(1, -1)]
    eb_spec = lambda d: pl.BlockSpec((EB, d), lambda g: (g, 0))
    w_spec = lambda a: pl.BlockSpec(a.shape, lambda g: (0,) * a.ndim)
    specs = [eb_spec(32), eb_spec(32), eb_spec(32)]
    specs += [w_spec(o) for o in ops[3:]]
    return pl.pallas_call(
        _pred_body,
        grid=(E // EB,),
        in_specs=specs,
        out_specs=eb_spec(2),
        out_shape=jax.ShapeDtypeStruct((E, 2), jnp.float32),
        interpret=_INTERPRET,
    )(*ops)


def kernel(x, edge_index, e, xbatch, params):
    N = x.shape[0]
    E = e.shape[0]
    row = edge_index[0]
    col = edge_index[1]

    xbn = _xbn_call(x, params['bn_node_g'], params['bn_node_b'])
    est = _estats_call(e)
    e_mean = est[0] / E
    e_var = est[1] / E - e_mean * e_mean
    a_e = (params['bn_edge_g'] * lax.rsqrt(e_var + 1e-5)).reshape(1, -1)
    c_e = (params['bn_edge_b'] - e_mean * a_e[0]).reshape(1, -1)

    xcur = xbn
    ef = e
    dims = [(16, 10), (32, 32), (32, 32)]
    for i, (nin, ein) in enumerate(dims):
        L = params['mp%d' % i]
        src, tgt = _gather2_call(xcur, row, col)
        aff = (a_e, c_e) if i == 0 else None
        ef, msg = _edge_call(nin, ein, src, tgt, ef, aff, L)
        agg2 = _scatter_call(msg, col, N)
        xcur = _update_call(agg2, xcur, L['root_w'], L['conv_b'])

    src, tgt = _gather2_call(xcur, row, col)
    return _pred_call(src, tgt, ef, params['pred'])


# speedup vs baseline: 2.9695x; 2.9695x over previous
import functools

import jax
import jax.numpy as jnp
from jax import lax
from jax.experimental import pallas as pl
from jax.experimental.pallas import tpu as pltpu
from jax.experimental.pallas import tpu_sc as plsc

LEAK = 0.1
EOUT = 32
NOUT = 32
EB = 1000

_INTERPRET = False


def _lk(v):
    return jnp.where(v >= 0, v, LEAK * v)


def _xbn_body(x_ref, g_ref, b_ref, xbn_ref):
    x = x_ref[...]
    m = jnp.mean(x, axis=0, keepdims=True)
    v = jnp.mean((x - m) * (x - m), axis=0, keepdims=True)
    xbn_ref[...] = (x - m) * lax.rsqrt(v + 1e-5) * g_ref[...] + b_ref[...]


def _xbn_call(x, g, b):
    N, D = x.shape
    return pl.pallas_call(
        _xbn_body,
        out_shape=jax.ShapeDtypeStruct((N, D), jnp.float32),
        interpret=_INTERPRET,
    )(x, g.reshape(1, D), b.reshape(1, D))


def _estats_body(e_ref, est_ref):
    @pl.when(pl.program_id(0) == 0)
    def _():
        est_ref[...] = jnp.zeros_like(est_ref)

    ch = e_ref[...]
    s = jnp.sum(ch, axis=0, keepdims=True)
    ss = jnp.sum(ch * ch, axis=0, keepdims=True)
    est_ref[...] += jnp.concatenate([s, ss], axis=0)


def _estats_call(e):
    E, ein = e.shape
    CH = 10000
    return pl.pallas_call(
        _estats_body,
        grid=(E // CH,),
        in_specs=[pl.BlockSpec((CH, ein), lambda g: (g, 0))],
        out_specs=pl.BlockSpec((2, ein), lambda g: (0, 0)),
        out_shape=jax.ShapeDtypeStruct((2, ein), jnp.float32),
        interpret=_INTERPRET,
    )(e)


_NC, _NS = 2, 16
_NW = _NC * _NS
_CHK = 128


def _gather_body(EPW, D, table_ref, row_ref, col_ref, src_ref, tgt_ref,
                 idx_a, buf_a, idx_b, buf_b, sem_a, sem_b):
    wid = lax.axis_index("s") * _NC + lax.axis_index("c")
    base = wid * EPW
    nch = (EPW + _CHK - 1) // _CHK

    def body(k, carry):
        off = base + jnp.minimum(k * _CHK, EPW - _CHK)
        pltpu.sync_copy(row_ref.at[pl.ds(off, _CHK)], idx_a)
        pltpu.sync_copy(col_ref.at[pl.ds(off, _CHK)], idx_b)
        ca = pltpu.async_copy(table_ref.at[idx_a], buf_a, sem_a)
        cb = pltpu.async_copy(table_ref.at[idx_b], buf_b, sem_b)
        ca.wait()
        pltpu.sync_copy(buf_a, src_ref.at[pl.ds(off, _CHK)])
        cb.wait()
        pltpu.sync_copy(buf_b, tgt_ref.at[pl.ds(off, _CHK)])
        return carry

    lax.fori_loop(0, nch, body, 0)


def _gather2_call(table, row, col):
    N, D = table.shape
    E = row.shape[0]
    EPW = E // _NW
    mesh = plsc.VectorSubcoreMesh(core_axis_name="c", subcore_axis_name="s")
    kfn = pl.kernel(
        functools.partial(_gather_body, EPW, D),
        out_type=(
            jax.ShapeDtypeStruct((E, D), jnp.float32),
            jax.ShapeDtypeStruct((E, D), jnp.float32),
        ),
        mesh=mesh,
        compiler_params=pltpu.CompilerParams(use_tc_tiling_on_sc=False),
        scratch_types=[
            pltpu.VMEM((_CHK,), jnp.int32),
            pltpu.VMEM((_CHK, D), jnp.float32),
            pltpu.VMEM((_CHK,), jnp.int32),
            pltpu.VMEM((_CHK, D), jnp.float32),
            pltpu.SemaphoreType.DMA,
            pltpu.SemaphoreType.DMA,
        ],
    )
    return kfn(table, row, col)


def _scatter_body(N, NFULL, msg_ref, col_ref, zeros_ref, agg_ref,
                  idx_v, buf_v, shared):
    cid = lax.axis_index("c")
    sid = lax.axis_index("s")
    wid = sid * _NC + cid

    @pl.when(sid == 0)
    def _():
        pltpu.sync_copy(zeros_ref, shared)

    plsc.subcore_barrier()

    def one_chunk(off):
        pltpu.sync_copy(col_ref.at[pl.ds(off, _CHK)], idx_v)
        pltpu.sync_copy(msg_ref.at[pl.ds(off, _CHK)], buf_v)
        pltpu.sync_copy(buf_v, shared.at[idx_v], add=True)

    per_w = NFULL // _NW
    base = wid * per_w * _CHK

    def body(k, carry):
        one_chunk(base + k * _CHK)
        return carry

    lax.fori_loop(0, per_w, body, 0)
    tail = NFULL - per_w * _NW

    @pl.when(wid < tail)
    def _():
        one_chunk((per_w * _NW + wid) * _CHK)

    plsc.subcore_barrier()
    rows = N // _NS
    pltpu.sync_copy(shared.at[pl.ds(sid * rows, rows)],
                    agg_ref.at[cid, pl.ds(sid * rows, rows)])


def _scatter_call(msg, col, N):
    E, D = msg.shape
    nfull = E // _CHK
    zeros = jnp.zeros((N, D), jnp.float32)
    mesh = plsc.VectorSubcoreMesh(core_axis_name="c", subcore_axis_name="s")
    kfn = pl.kernel(
        functools.partial(_scatter_body, N, nfull),
        out_type=jax.ShapeDtypeStruct((_NC, N, D), jnp.float32),
        mesh=mesh,
        compiler_params=pltpu.CompilerParams(use_tc_tiling_on_sc=False),
        scratch_types=[
            pltpu.VMEM((_CHK,), jnp.int32),
            pltpu.VMEM((_CHK, D), jnp.float32),
            pltpu.VMEM_SHARED((N, D), jnp.float32),
        ],
    )
    return kfn(msg, col, zeros)


def _fold32(p):
    B, w = p.shape
    q = p[:, 0:min(128, w)]
    if q.shape[1] < 128:
        q = jnp.concatenate(
            [q, jnp.zeros((B, 128 - q.shape[1]), jnp.float32)], axis=1)
    c = 128
    while c + 128 <= w:
        q = q + p[:, c:c + 128]
        c += 128
    if c < w:
        r = w - c
        q = q + jnp.concatenate(
            [p[:, c:w], jnp.zeros((B, 128 - r), jnp.float32)], axis=1)
    h = q[:, 0:64] + q[:, 64:128]
    return h[:, 0:32] + h[:, 32:64]


def _edge_body(nin, ein, has_aff, *refs):
    if has_aff:
        (src_ref, tgt_ref, e_ref, a_ref, c_ref, r1_ref, r2_ref,
         c1_ref, c2_ref, beeb_ref,
         w1_ref, b1_ref, w2_ref, b2_ref, enew_ref, msg_ref) = refs
    else:
        (src_ref, tgt_ref, e_ref, r1_ref, r2_ref,
         c1_ref, c2_ref, beeb_ref,
         w1_ref, b1_ref, w2_ref, b2_ref, enew_ref, msg_ref) = refs
    dot = lambda a, b: jnp.dot(a, b, preferred_element_type=jnp.float32)
    s = src_ref[...]
    t = tgt_ref[...]
    ef = e_ref[...]
    if has_aff:
        ef = ef * a_ref[...] + c_ref[...]
    s_exp = dot(s, r1_ref[...])
    a1 = dot(t, c1_ref[...])
    z = _fold32(s_exp * a1)
    e_exp = dot(ef, r2_ref[...])
    a2 = dot(z, c2_ref[...])
    z2 = _fold32(e_exp * a2)
    en = _lk(z2 + beeb_ref[...])
    enew_ref[...] = en
    h1 = _lk(dot(en, w1_ref[...]) + b1_ref[...])
    h2 = _lk(dot(h1, w2_ref[...]) + b2_ref[...])
    msg_ref[...] = _fold32(s_exp * h2)


def _edge_call(nin, ein, src, tgt, ef, aff, L):
    E = src.shape[0]
    c1 = L['bst_w'].transpose(2, 1, 0).reshape(nin, nin * EOUT)
    c2 = L['bee_w'].transpose(1, 2, 0).reshape(EOUT, ein * EOUT)
    r1 = (jnp.arange(nin * EOUT) // EOUT
          == jnp.arange(nin)[:, None]).astype(jnp.float32)
    r2 = (jnp.arange(ein * EOUT) // EOUT
          == jnp.arange(ein)[:, None]).astype(jnp.float32)
    beeb = L['bee_b'].reshape(1, EOUT)
    w1 = L['nn1_w'].T
    b1 = L['nn1_b'].reshape(1, nin)
    w2 = L['nn2_w'].T
    b2 = L['nn2_b'].reshape(1, nin * NOUT)
    grid = (E // EB,)
    eb_spec = lambda d: pl.BlockSpec((EB, d), lambda g: (g, 0))
    w_spec = lambda a: pl.BlockSpec(a.shape, lambda g: (0,) * a.ndim)
    ops = [src, tgt, ef]
    specs = [eb_spec(nin), eb_spec(nin), eb_spec(ein)]
    if aff is not None:
        ops += [aff[0], aff[1]]
        specs += [w_spec(aff[0]), w_spec(aff[1])]
    ops += [r1, r2, c1, c2, beeb, w1, b1, w2, b2]
    specs += [w_spec(o) for o in (r1, r2, c1, c2, beeb, w1, b1, w2, b2)]
    return pl.pallas_call(
        functools.partial(_edge_body, nin, ein, aff is not None),
        grid=grid,
        in_specs=specs,
        out_specs=(eb_spec(EOUT), eb_spec(NOUT)),
        out_shape=(
            jax.ShapeDtypeStruct((E, EOUT), jnp.float32),
            jax.ShapeDtypeStruct((E, NOUT), jnp.float32),
        ),
        interpret=_INTERPRET,
    )(*ops)


def _update_body(agg_ref, x_ref, rw_ref, cb_ref, out_ref):
    out_ref[...] = (agg_ref[0] + agg_ref[1]
                    + jnp.dot(x_ref[...], rw_ref[...],
                              preferred_element_type=jnp.float32)
                    + cb_ref[...])


def _update_call(agg2, x, rw, cb):
    N = x.shape[0]
    return pl.pallas_call(
        _update_body,
        out_shape=jax.ShapeDtypeStruct((N, NOUT), jnp.float32),
        interpret=_INTERPRET,
    )(agg2, x, rw.T, cb.reshape(1, NOUT))


def _pred_body(s_ref, t_ref, e_ref, w0s_ref, w0t_ref, w0e_ref, b0_ref,
               w1_ref, b1_ref, w2_ref, b2_ref, w3_ref, b3_ref,
               w4_ref, b4_ref, out_ref):
    dot = lambda a, b: jnp.dot(a, b, preferred_element_type=jnp.float32)
    h = (dot(s_ref[...], w0s_ref[...]) + dot(t_ref[...], w0t_ref[...])
         + dot(e_ref[...], w0e_ref[...]) + b0_ref[...])
    h = _lk(h)
    h = _lk(dot(h, w1_ref[...]) + b1_ref[...])
    h = _lk(dot(h, w2_ref[...]) + b2_ref[...])
    h = _lk(dot(h, w3_ref[...]) + b3_ref[...])
    out_ref[...] = dot(h, w4_ref[...]) + b4_ref[...]


def _pred_call(src, tgt, ef, P):
    E = src.shape[0]
    w0 = P['w0']
    ops = [src, tgt, ef,
           w0[:, :32].T, w0[:, 32:64].T, w0[:, 64:96].T, P['b0'].reshape(1, -1),
           P['w1'].T, P['b1'].reshape(1, -1),
           P['w2'].T, P['b2'].reshape(1, -1),
           P['w3'].T, P['b3'].reshape(1, -1),
           P['w4'].T, P['b4'].reshape(1, -1)]
    eb_spec = lambda d: pl.BlockSpec((EB, d), lambda g: (g, 0))
    w_spec = lambda a: pl.BlockSpec(a.shape, lambda g: (0,) * a.ndim)
    specs = [eb_spec(32), eb_spec(32), eb_spec(32)]
    specs += [w_spec(o) for o in ops[3:]]
    return pl.pallas_call(
        _pred_body,
        grid=(E // EB,),
        in_specs=specs,
        out_specs=eb_spec(2),
        out_shape=jax.ShapeDtypeStruct((E, 2), jnp.float32),
        interpret=_INTERPRET,
    )(*ops)


def kernel(x, edge_index, e, xbatch, params):
    N = x.shape[0]
    E = e.shape[0]
    row = edge_index[0]
    col = edge_index[1]

    xbn = _xbn_call(x, params['bn_node_g'], params['bn_node_b'])
    est = _estats_call(e)
    e_mean = est[0] / E
    e_var = est[1] / E - e_mean * e_mean
    a_e = (params['bn_edge_g'] * lax.rsqrt(e_var + 1e-5)).reshape(1, -1)
    c_e = (params['bn_edge_b'] - e_mean * a_e[0]).reshape(1, -1)

    xcur = xbn
    ef = e
    dims = [(16, 10), (32, 32), (32, 32)]
    for i, (nin, ein) in enumerate(dims):
        L = params['mp%d' % i]
        src, tgt = _gather2_call(xcur, row, col)
        aff = (a_e, c_e) if i == 0 else None
        ef, msg = _edge_call(nin, ein, src, tgt, ef, aff, L)
        agg2 = _scatter_call(msg, col, N)
        xcur = _update_call(agg2, xcur, L['root_w'], L['conv_b'])

    src, tgt = _gather2_call(xcur, row, col)
    return _pred_call(src, tgt, ef, params['pred'])


# speedup vs baseline: 3.6017x; 1.2129x over previous
import functools

import jax
import jax.numpy as jnp
from jax import lax
from jax.experimental import pallas as pl
from jax.experimental.pallas import tpu as pltpu
from jax.experimental.pallas import tpu_sc as plsc

LEAK = 0.1
EOUT = 32
NOUT = 32
EB = 2000

_INTERPRET = False


def _lk(v):
    return jnp.where(v >= 0, v, LEAK * v)


def _xbn_body(x_ref, g_ref, b_ref, xbn_ref):
    x = x_ref[...]
    m = jnp.mean(x, axis=0, keepdims=True)
    v = jnp.mean((x - m) * (x - m), axis=0, keepdims=True)
    xbn_ref[...] = (x - m) * lax.rsqrt(v + 1e-5) * g_ref[...] + b_ref[...]


def _xbn_call(x, g, b):
    N, D = x.shape
    return pl.pallas_call(
        _xbn_body,
        out_shape=jax.ShapeDtypeStruct((N, D), jnp.float32),
        interpret=_INTERPRET,
    )(x, g.reshape(1, D), b.reshape(1, D))


def _estats_body(e_ref, est_ref):
    @pl.when(pl.program_id(0) == 0)
    def _():
        est_ref[...] = jnp.zeros_like(est_ref)

    ch = e_ref[...]
    s = jnp.sum(ch, axis=0, keepdims=True)
    ss = jnp.sum(ch * ch, axis=0, keepdims=True)
    est_ref[...] += jnp.concatenate([s, ss], axis=0)


def _estats_call(e):
    E, ein = e.shape
    CH = 10000
    return pl.pallas_call(
        _estats_body,
        grid=(E // CH,),
        in_specs=[pl.BlockSpec((CH, ein), lambda g: (g, 0))],
        out_specs=pl.BlockSpec((2, ein), lambda g: (0, 0)),
        out_shape=jax.ShapeDtypeStruct((2, ein), jnp.float32),
        interpret=_INTERPRET,
    )(e)


_NC, _NS = 2, 16
_NW = _NC * _NS
_CHK = 128


_GRP = 4
_GW = _GRP * _CHK


def _gather_body(EPW, D, table_ref, row_ref, col_ref, src_ref, tgt_ref,
                 ridx, cidx, bufa, bufb, sem0, sem1):
    wid = lax.axis_index("s") * _NC + lax.axis_index("c")
    base = wid * EPW
    pltpu.sync_copy(row_ref.at[pl.ds(base, EPW)], ridx)
    pltpu.sync_copy(col_ref.at[pl.ds(base, EPW)], cidx)
    ngrp = (EPW + _GW - 1) // _GW
    offs = [min(g * _GW, EPW - _GW) for g in range(ngrp)]
    sems = (sem0, sem1)
    pending = {}

    def issue(g):
        slot = g % 2
        cps = []
        for j in range(_GRP):
            lo = offs[g] + j * _CHK
            cps.append(pltpu.async_copy(
                table_ref.at[ridx.at[pl.ds(lo, _CHK)]],
                bufa.at[slot, pl.ds(j * _CHK, _CHK)], sems[slot]))
            cps.append(pltpu.async_copy(
                table_ref.at[cidx.at[pl.ds(lo, _CHK)]],
                bufb.at[slot, pl.ds(j * _CHK, _CHK)], sems[slot]))
        pending[g] = cps

    def drain(g):
        slot = g % 2
        for cp in pending.pop(g):
            cp.wait()
        pltpu.sync_copy(bufa.at[slot], src_ref.at[pl.ds(base + offs[g], _GW)])
        pltpu.sync_copy(bufb.at[slot], tgt_ref.at[pl.ds(base + offs[g], _GW)])

    issue(0)
    for g in range(1, ngrp):
        issue(g)
        drain(g - 1)
    drain(ngrp - 1)


def _gather2_call(table, row, col):
    N, D = table.shape
    E = row.shape[0]
    EPW = E // _NW
    mesh = plsc.VectorSubcoreMesh(core_axis_name="c", subcore_axis_name="s")
    kfn = pl.kernel(
        functools.partial(_gather_body, EPW, D),
        out_type=(
            jax.ShapeDtypeStruct((E, D), jnp.float32),
            jax.ShapeDtypeStruct((E, D), jnp.float32),
        ),
        mesh=mesh,
        compiler_params=pltpu.CompilerParams(use_tc_tiling_on_sc=False),
        scratch_types=[
            pltpu.VMEM((EPW,), jnp.int32),
            pltpu.VMEM((EPW,), jnp.int32),
            pltpu.VMEM((2, _GW, D), jnp.float32),
            pltpu.VMEM((2, _GW, D), jnp.float32),
            pltpu.SemaphoreType.DMA,
            pltpu.SemaphoreType.DMA,
        ],
    )
    return kfn(table, row, col)


def _scatter_body(N, NFULL, msg_ref, col_ref, zeros_ref, agg_ref,
                  idx2, buf2, shared, sem0, sem1):
    cid = lax.axis_index("c")
    sid = lax.axis_index("s")
    wid = sid * _NC + cid

    @pl.when(sid == 0)
    def _():
        pltpu.sync_copy(zeros_ref, shared)

    plsc.subcore_barrier()

    per_w = NFULL // _NW
    base = wid * per_w * _CHK
    sems = (sem0, sem1)

    def issue(k):
        slot = k % 2
        ci = pltpu.async_copy(col_ref.at[pl.ds(base + k * _CHK, _CHK)],
                              idx2.at[slot], sems[slot])
        cm = pltpu.async_copy(msg_ref.at[pl.ds(base + k * _CHK, _CHK)],
                              buf2.at[slot], sems[slot])
        return (ci, cm)

    pend = issue(0)
    for k in range(per_w):
        nxt = issue(k + 1) if k + 1 < per_w else None
        for cp in pend:
            cp.wait()
        slot = k % 2
        pltpu.sync_copy(buf2.at[slot], shared.at[idx2.at[slot]], add=True)
        pend = nxt

    tail = NFULL - per_w * _NW

    @pl.when(wid < tail)
    def _():
        off = per_w * _NW * _CHK + wid * _CHK
        pltpu.sync_copy(col_ref.at[pl.ds(off, _CHK)], idx2.at[0])
        pltpu.sync_copy(msg_ref.at[pl.ds(off, _CHK)], buf2.at[0])
        pltpu.sync_copy(buf2.at[0], shared.at[idx2.at[0]], add=True)

    plsc.subcore_barrier()
    rows = N // _NS
    pltpu.sync_copy(shared.at[pl.ds(sid * rows, rows)],
                    agg_ref.at[cid, pl.ds(sid * rows, rows)])


def _scatter_call(msg, col, N):
    E, D = msg.shape
    nfull = E // _CHK
    zeros = jnp.zeros((N, D), jnp.float32)
    mesh = plsc.VectorSubcoreMesh(core_axis_name="c", subcore_axis_name="s")
    kfn = pl.kernel(
        functools.partial(_scatter_body, N, nfull),
        out_type=jax.ShapeDtypeStruct((_NC, N, D), jnp.float32),
        mesh=mesh,
        compiler_params=pltpu.CompilerParams(use_tc_tiling_on_sc=False),
        scratch_types=[
            pltpu.VMEM((2, _CHK), jnp.int32),
            pltpu.VMEM((2, _CHK, D), jnp.float32),
            pltpu.VMEM_SHARED((N, D), jnp.float32),
            pltpu.SemaphoreType.DMA,
            pltpu.SemaphoreType.DMA,
        ],
    )
    return kfn(msg, col, zeros)


def _fold32(p):
    B, w = p.shape
    q = p[:, 0:min(128, w)]
    if q.shape[1] < 128:
        q = jnp.concatenate(
            [q, jnp.zeros((B, 128 - q.shape[1]), jnp.float32)], axis=1)
    c = 128
    while c + 128 <= w:
        q = q + p[:, c:c + 128]
        c += 128
    if c < w:
        r = w - c
        q = q + jnp.concatenate(
            [p[:, c:w], jnp.zeros((B, 128 - r), jnp.float32)], axis=1)
    h = q[:, 0:64] + q[:, 64:128]
    return h[:, 0:32] + h[:, 32:64]


def _edge_body(nin, ein, has_aff, *refs):
    if has_aff:
        (src_ref, tgt_ref, e_ref, a_ref, c_ref, r1_ref, r2_ref,
         c1_ref, c2_ref, beeb_ref,
         w1_ref, b1_ref, w2_ref, b2_ref, enew_ref, msg_ref) = refs
    else:
        (src_ref, tgt_ref, e_ref, r1_ref, r2_ref,
         c1_ref, c2_ref, beeb_ref,
         w1_ref, b1_ref, w2_ref, b2_ref, enew_ref, msg_ref) = refs
    dot = lambda a, b: jnp.dot(a, b, preferred_element_type=jnp.float32)
    s = src_ref[...]
    t = tgt_ref[...]
    ef = e_ref[...]
    if has_aff:
        ef = ef * a_ref[...] + c_ref[...]
    s_exp = dot(s, r1_ref[...])
    a1 = dot(t, c1_ref[...])
    z = _fold32(s_exp * a1)
    e_exp = dot(ef, r2_ref[...])
    a2 = dot(z, c2_ref[...])
    z2 = _fold32(e_exp * a2)
    en = _lk(z2 + beeb_ref[...])
    enew_ref[...] = en
    h1 = _lk(dot(en, w1_ref[...]) + b1_ref[...])
    h2 = _lk(dot(h1, w2_ref[...]) + b2_ref[...])
    msg_ref[...] = _fold32(s_exp * h2)


def _edge_call(nin, ein, src, tgt, ef, aff, L):
    E = src.shape[0]
    c1 = L['bst_w'].transpose(2, 1, 0).reshape(nin, nin * EOUT)
    c2 = L['bee_w'].transpose(1, 2, 0).reshape(EOUT, ein * EOUT)
    r1 = (jnp.arange(nin * EOUT) // EOUT
          == jnp.arange(nin)[:, None]).astype(jnp.float32)
    r2 = (jnp.arange(ein * EOUT) // EOUT
          == jnp.arange(ein)[:, None]).astype(jnp.float32)
    beeb = L['bee_b'].reshape(1, EOUT)
    w1 = L['nn1_w'].T
    b1 = L['nn1_b'].reshape(1, nin)
    w2 = L['nn2_w'].T
    b2 = L['nn2_b'].reshape(1, nin * NOUT)
    grid = (E // EB,)
    eb_spec = lambda d: pl.BlockSpec((EB, d), lambda g: (g, 0))
    w_spec = lambda a: pl.BlockSpec(a.shape, lambda g: (0,) * a.ndim)
    ops = [src, tgt, ef]
    specs = [eb_spec(nin), eb_spec(nin), eb_spec(ein)]
    if aff is not None:
        ops += [aff[0], aff[1]]
        specs += [w_spec(aff[0]), w_spec(aff[1])]
    ops += [r1, r2, c1, c2, beeb, w1, b1, w2, b2]
    specs += [w_spec(o) for o in (r1, r2, c1, c2, beeb, w1, b1, w2, b2)]
    return pl.pallas_call(
        functools.partial(_edge_body, nin, ein, aff is not None),
        grid=grid,
        in_specs=specs,
        out_specs=(eb_spec(EOUT), eb_spec(NOUT)),
        out_shape=(
            jax.ShapeDtypeStruct((E, EOUT), jnp.float32),
            jax.ShapeDtypeStruct((E, NOUT), jnp.float32),
        ),
        interpret=_INTERPRET,
    )(*ops)


def _update_body(agg_ref, x_ref, rw_ref, cb_ref, out_ref):
    out_ref[...] = (agg_ref[0] + agg_ref[1]
                    + jnp.dot(x_ref[...], rw_ref[...],
                              preferred_element_type=jnp.float32)
                    + cb_ref[...])


def _update_call(agg2, x, rw, cb):
    N = x.shape[0]
    return pl.pallas_call(
        _update_body,
        out_shape=jax.ShapeDtypeStruct((N, NOUT), jnp.float32),
        interpret=_INTERPRET,
    )(agg2, x, rw.T, cb.reshape(1, NOUT))


def _pred_body(s_ref, t_ref, e_ref, w0s_ref, w0t_ref, w0e_ref, b0_ref,
               w1_ref, b1_ref, w2_ref, b2_ref, w3_ref, b3_ref,
               w4_ref, b4_ref, out_ref):
    dot = lambda a, b: jnp.dot(a, b, preferred_element_type=jnp.float32)
    h = (dot(s_ref[...], w0s_ref[...]) + dot(t_ref[...], w0t_ref[...])
         + dot(e_ref[...], w0e_ref[...]) + b0_ref[...])
    h = _lk(h)
    h = _lk(dot(h, w1_ref[...]) + b1_ref[...])
    h = _lk(dot(h, w2_ref[...]) + b2_ref[...])
    h = _lk(dot(h, w3_ref[...]) + b3_ref[...])
    out_ref[...] = dot(h, w4_ref[...]) + b4_ref[...]


def _pred_call(src, tgt, ef, P):
    E = src.shape[0]
    w0 = P['w0']
    ops = [src, tgt, ef,
           w0[:, :32].T, w0[:, 32:64].T, w0[:, 64:96].T, P['b0'].reshape(1, -1),
           P['w1'].T, P['b1'].reshape(1, -1),
           P['w2'].T, P['b2'].reshape(1, -1),
           P['w3'].T, P['b3'].reshape(1, -1),
           P['w4'].T, P['b4'].reshape(1, -1)]
    eb_spec = lambda d: pl.BlockSpec((EB, d), lambda g: (g, 0))
    w_spec = lambda a: pl.BlockSpec(a.shape, lambda g: (0,) * a.ndim)
    specs = [eb_spec(32), eb_spec(32), eb_spec(32)]
    specs += [w_spec(o) for o in ops[3:]]
    return pl.pallas_call(
        _pred_body,
        grid=(E // EB,),
        in_specs=specs,
        out_specs=eb_spec(2),
        out_shape=jax.ShapeDtypeStruct((E, 2), jnp.float32),
        interpret=_INTERPRET,
    )(*ops)


def kernel(x, edge_index, e, xbatch, params):
    N = x.shape[0]
    E = e.shape[0]
    row = edge_index[0]
    col = edge_index[1]

    xbn = _xbn_call(x, params['bn_node_g'], params['bn_node_b'])
    est = _estats_call(e)
    e_mean = est[0] / E
    e_var = est[1] / E - e_mean * e_mean
    a_e = (params['bn_edge_g'] * lax.rsqrt(e_var + 1e-5)).reshape(1, -1)
    c_e = (params['bn_edge_b'] - e_mean * a_e[0]).reshape(1, -1)

    xcur = xbn
    ef = e
    dims = [(16, 10), (32, 32), (32, 32)]
    for i, (nin, ein) in enumerate(dims):
        L = params['mp%d' % i]
        src, tgt = _gather2_call(xcur, row, col)
        aff = (a_e, c_e) if i == 0 else None
        ef, msg = _edge_call(nin, ein, src, tgt, ef, aff, L)
        agg2 = _scatter_call(msg, col, N)
        xcur = _update_call(agg2, xcur, L['root_w'], L['conv_b'])

    src, tgt = _gather2_call(xcur, row, col)
    return _pred_call(src, tgt, ef, params['pred'])


# speedup vs baseline: 3.6878x; 1.0239x over previous
import functools

import jax
import jax.numpy as jnp
from jax import lax
from jax.experimental import pallas as pl
from jax.experimental.pallas import tpu as pltpu
from jax.experimental.pallas import tpu_sc as plsc

LEAK = 0.1
EOUT = 32
NOUT = 32
EB = 4000

_INTERPRET = False


def _lk(v):
    return jnp.where(v >= 0, v, LEAK * v)


def _stats_body(x_ref, e_ref, g_ref, b_ref, xbn_ref, est_ref):
    @pl.when(pl.program_id(0) == 0)
    def _():
        x = x_ref[...]
        m = jnp.mean(x, axis=0, keepdims=True)
        v = jnp.mean((x - m) * (x - m), axis=0, keepdims=True)
        xbn_ref[...] = ((x - m) * lax.rsqrt(v + 1e-5) * g_ref[...]
                        + b_ref[...])
        est_ref[...] = jnp.zeros_like(est_ref)

    ch = e_ref[...]
    s = jnp.sum(ch, axis=0, keepdims=True)
    ss = jnp.sum(ch * ch, axis=0, keepdims=True)
    est_ref[...] += jnp.concatenate([s, ss], axis=0)


def _stats_call(x, e, g, b):
    N, D = x.shape
    E, ein = e.shape
    CH = 10000
    full = lambda a: pl.BlockSpec(a.shape, lambda i: (0,) * a.ndim)
    g2 = g.reshape(1, D)
    b2 = b.reshape(1, D)
    return pl.pallas_call(
        _stats_body,
        grid=(E // CH,),
        in_specs=[full(x), pl.BlockSpec((CH, ein), lambda i: (i, 0)),
                  full(g2), full(b2)],
        out_specs=(pl.BlockSpec((N, D), lambda i: (0, 0)),
                   pl.BlockSpec((2, ein), lambda i: (0, 0))),
        out_shape=(jax.ShapeDtypeStruct((N, D), jnp.float32),
                   jax.ShapeDtypeStruct((2, ein), jnp.float32)),
        interpret=_INTERPRET,
    )(x, e, g2, b2)


_NC, _NS = 2, 16
_NW = _NC * _NS
_CHK = 128


_GRP = 4
_GW = _GRP * _CHK


def _gather_body(EPW, D, table_ref, row_ref, col_ref, src_ref, tgt_ref,
                 ridx, cidx, bufa, bufb, sem0, sem1):
    wid = lax.axis_index("s") * _NC + lax.axis_index("c")
    base = wid * EPW
    pltpu.sync_copy(row_ref.at[pl.ds(base, EPW)], ridx)
    pltpu.sync_copy(col_ref.at[pl.ds(base, EPW)], cidx)
    ngrp = (EPW + _GW - 1) // _GW
    offs = [min(g * _GW, EPW - _GW) for g in range(ngrp)]
    sems = (sem0, sem1)
    pending = {}

    def issue(g):
        slot = g % 2
        cps = []
        for j in range(_GRP):
            lo = offs[g] + j * _CHK
            cps.append(pltpu.async_copy(
                table_ref.at[ridx.at[pl.ds(lo, _CHK)]],
                bufa.at[slot, pl.ds(j * _CHK, _CHK)], sems[slot]))
            cps.append(pltpu.async_copy(
                table_ref.at[cidx.at[pl.ds(lo, _CHK)]],
                bufb.at[slot, pl.ds(j * _CHK, _CHK)], sems[slot]))
        pending[g] = cps

    def drain(g):
        slot = g % 2
        for cp in pending.pop(g):
            cp.wait()
        pltpu.sync_copy(bufa.at[slot], src_ref.at[pl.ds(base + offs[g], _GW)])
        pltpu.sync_copy(bufb.at[slot], tgt_ref.at[pl.ds(base + offs[g], _GW)])

    issue(0)
    for g in range(1, ngrp):
        issue(g)
        drain(g - 1)
    drain(ngrp - 1)


def _gather2_call(table, row, col):
    N, D = table.shape
    E = row.shape[0]
    EPW = E // _NW
    mesh = plsc.VectorSubcoreMesh(core_axis_name="c", subcore_axis_name="s")
    kfn = pl.kernel(
        functools.partial(_gather_body, EPW, D),
        out_type=(
            jax.ShapeDtypeStruct((E, D), jnp.float32),
            jax.ShapeDtypeStruct((E, D), jnp.float32),
        ),
        mesh=mesh,
        compiler_params=pltpu.CompilerParams(use_tc_tiling_on_sc=False),
        scratch_types=[
            pltpu.VMEM((EPW,), jnp.int32),
            pltpu.VMEM((EPW,), jnp.int32),
            pltpu.VMEM((2, _GW, D), jnp.float32),
            pltpu.VMEM((2, _GW, D), jnp.float32),
            pltpu.SemaphoreType.DMA,
            pltpu.SemaphoreType.DMA,
        ],
    )
    return kfn(table, row, col)


def _scatter_body(N, NFULL, msg_ref, col_ref, zeros_ref, agg_ref,
                  idx2, buf2, shared, sem0, sem1):
    cid = lax.axis_index("c")
    sid = lax.axis_index("s")
    wid = sid * _NC + cid

    @pl.when(sid == 0)
    def _():
        pltpu.sync_copy(zeros_ref, shared)

    plsc.subcore_barrier()

    per_w = NFULL // _NW
    base = wid * per_w * _CHK
    sems = (sem0, sem1)

    def issue(k):
        slot = k % 2
        ci = pltpu.async_copy(col_ref.at[pl.ds(base + k * _CHK, _CHK)],
                              idx2.at[slot], sems[slot])
        cm = pltpu.async_copy(msg_ref.at[pl.ds(base + k * _CHK, _CHK)],
                              buf2.at[slot], sems[slot])
        return (ci, cm)

    pend = issue(0)
    for k in range(per_w):
        nxt = issue(k + 1) if k + 1 < per_w else None
        for cp in pend:
            cp.wait()
        slot = k % 2
        pltpu.sync_copy(buf2.at[slot], shared.at[idx2.at[slot]], add=True)
        pend = nxt

    tail = NFULL - per_w * _NW

    @pl.when(wid < tail)
    def _():
        off = per_w * _NW * _CHK + wid * _CHK
        pltpu.sync_copy(col_ref.at[pl.ds(off, _CHK)], idx2.at[0])
        pltpu.sync_copy(msg_ref.at[pl.ds(off, _CHK)], buf2.at[0])
        pltpu.sync_copy(buf2.at[0], shared.at[idx2.at[0]], add=True)

    plsc.subcore_barrier()
    rows = N // _NS
    pltpu.sync_copy(shared.at[pl.ds(sid * rows, rows)],
                    agg_ref.at[cid, pl.ds(sid * rows, rows)])


def _scatter_call(msg, col, N):
    E, D = msg.shape
    nfull = E // _CHK
    zeros = jnp.zeros((N, D), jnp.float32)
    mesh = plsc.VectorSubcoreMesh(core_axis_name="c", subcore_axis_name="s")
    kfn = pl.kernel(
        functools.partial(_scatter_body, N, nfull),
        out_type=jax.ShapeDtypeStruct((_NC, N, D), jnp.float32),
        mesh=mesh,
        compiler_params=pltpu.CompilerParams(use_tc_tiling_on_sc=False),
        scratch_types=[
            pltpu.VMEM((2, _CHK), jnp.int32),
            pltpu.VMEM((2, _CHK, D), jnp.float32),
            pltpu.VMEM_SHARED((N, D), jnp.float32),
            pltpu.SemaphoreType.DMA,
            pltpu.SemaphoreType.DMA,
        ],
    )
    return kfn(msg, col, zeros)


def _fold32(p):
    B, w = p.shape
    q = p[:, 0:min(128, w)]
    if q.shape[1] < 128:
        q = jnp.concatenate(
            [q, jnp.zeros((B, 128 - q.shape[1]), jnp.float32)], axis=1)
    c = 128
    while c + 128 <= w:
        q = q + p[:, c:c + 128]
        c += 128
    if c < w:
        r = w - c
        q = q + jnp.concatenate(
            [p[:, c:w], jnp.zeros((B, 128 - r), jnp.float32)], axis=1)
    h = q[:, 0:64] + q[:, 64:128]
    return h[:, 0:32] + h[:, 32:64]


def _edge_body(nin, ein, has_aff, *refs):
    if has_aff:
        (src_ref, tgt_ref, e_ref, a_ref, c_ref, r1_ref, r2_ref,
         c1_ref, c2_ref, beeb_ref,
         w1_ref, b1_ref, w2_ref, b2_ref, enew_ref, msg_ref) = refs
    else:
        (src_ref, tgt_ref, e_ref, r1_ref, r2_ref,
         c1_ref, c2_ref, beeb_ref,
         w1_ref, b1_ref, w2_ref, b2_ref, enew_ref, msg_ref) = refs
    dot = lambda a, b: jnp.dot(a, b, preferred_element_type=jnp.float32)
    s = src_ref[...]
    t = tgt_ref[...]
    ef = e_ref[...]
    if has_aff:
        ef = ef * a_ref[...] + c_ref[...]
    s_exp = dot(s, r1_ref[...])
    a1 = dot(t, c1_ref[...])
    z = _fold32(s_exp * a1)
    e_exp = dot(ef, r2_ref[...])
    a2 = dot(z, c2_ref[...])
    z2 = _fold32(e_exp * a2)
    en = _lk(z2 + beeb_ref[...])
    enew_ref[...] = en
    h1 = _lk(dot(en, w1_ref[...]) + b1_ref[...])
    h2 = _lk(dot(h1, w2_ref[...]) + b2_ref[...])
    msg_ref[...] = _fold32(s_exp * h2)


def _edge_call(nin, ein, src, tgt, ef, aff, L):
    E = src.shape[0]
    c1 = L['bst_w'].transpose(2, 1, 0).reshape(nin, nin * EOUT)
    c2 = L['bee_w'].transpose(1, 2, 0).reshape(EOUT, ein * EOUT)
    r1 = (jnp.arange(nin * EOUT) // EOUT
          == jnp.arange(nin)[:, None]).astype(jnp.float32)
    r2 = (jnp.arange(ein * EOUT) // EOUT
          == jnp.arange(ein)[:, None]).astype(jnp.float32)
    beeb = L['bee_b'].reshape(1, EOUT)
    w1 = L['nn1_w'].T
    b1 = L['nn1_b'].reshape(1, nin)
    w2 = L['nn2_w'].T
    b2 = L['nn2_b'].reshape(1, nin * NOUT)
    grid = (E // EB,)
    eb_spec = lambda d: pl.BlockSpec((EB, d), lambda g: (g, 0))
    w_spec = lambda a: pl.BlockSpec(a.shape, lambda g: (0,) * a.ndim)
    ops = [src, tgt, ef]
    specs = [eb_spec(nin), eb_spec(nin), eb_spec(ein)]
    if aff is not None:
        ops += [aff[0], aff[1]]
        specs += [w_spec(aff[0]), w_spec(aff[1])]
    ops += [r1, r2, c1, c2, beeb, w1, b1, w2, b2]
    specs += [w_spec(o) for o in (r1, r2, c1, c2, beeb, w1, b1, w2, b2)]
    return pl.pallas_call(
        functools.partial(_edge_body, nin, ein, aff is not None),
        grid=grid,
        in_specs=specs,
        out_specs=(eb_spec(EOUT), eb_spec(NOUT)),
        out_shape=(
            jax.ShapeDtypeStruct((E, EOUT), jnp.float32),
            jax.ShapeDtypeStruct((E, NOUT), jnp.float32),
        ),
        interpret=_INTERPRET,
    )(*ops)


def _update_body(agg_ref, x_ref, rw_ref, cb_ref, out_ref):
    out_ref[...] = (agg_ref[0] + agg_ref[1]
                    + jnp.dot(x_ref[...], rw_ref[...],
                              preferred_element_type=jnp.float32)
                    + cb_ref[...])


def _update_call(agg2, x, rw, cb):
    N = x.shape[0]
    return pl.pallas_call(
        _update_body,
        out_shape=jax.ShapeDtypeStruct((N, NOUT), jnp.float32),
        interpret=_INTERPRET,
    )(agg2, x, rw.T, cb.reshape(1, NOUT))


def _pred_body(s_ref, t_ref, e_ref, w0s_ref, w0t_ref, w0e_ref, b0_ref,
               w1_ref, b1_ref, w2_ref, b2_ref, w3_ref, b3_ref,
               w4_ref, b4_ref, out_ref):
    dot = lambda a, b: jnp.dot(a, b, preferred_element_type=jnp.float32)
    h = (dot(s_ref[...], w0s_ref[...]) + dot(t_ref[...], w0t_ref[...])
         + dot(e_ref[...], w0e_ref[...]) + b0_ref[...])
    h = _lk(h)
    h = _lk(dot(h, w1_ref[...]) + b1_ref[...])
    h = _lk(dot(h, w2_ref[...]) + b2_ref[...])
    h = _lk(dot(h, w3_ref[...]) + b3_ref[...])
    out_ref[...] = dot(h, w4_ref[...]) + b4_ref[...]


def _pred_call(src, tgt, ef, P):
    E = src.shape[0]
    w0 = P['w0']
    ops = [src, tgt, ef,
           w0[:, :32].T, w0[:, 32:64].T, w0[:, 64:96].T, P['b0'].reshape(1, -1),
           P['w1'].T, P['b1'].reshape(1, -1),
           P['w2'].T, P['b2'].reshape(1, -1),
           P['w3'].T, P['b3'].reshape(1, -1),
           P['w4'].T, P['b4'].reshape(1, -1)]
    eb_spec = lambda d: pl.BlockSpec((EB, d), lambda g: (g, 0))
    w_spec = lambda a: pl.BlockSpec(a.shape, lambda g: (0,) * a.ndim)
    specs = [eb_spec(32), eb_spec(32), eb_spec(32)]
    specs += [w_spec(o) for o in ops[3:]]
    return pl.pallas_call(
        _pred_body,
        grid=(E // EB,),
        in_specs=specs,
        out_specs=eb_spec(2),
        out_shape=jax.ShapeDtypeStruct((E, 2), jnp.float32),
        interpret=_INTERPRET,
    )(*ops)


def kernel(x, edge_index, e, xbatch, params):
    N = x.shape[0]
    E = e.shape[0]
    row = edge_index[0]
    col = edge_index[1]

    xbn, est = _stats_call(x, e, params['bn_node_g'], params['bn_node_b'])
    e_mean = est[0] / E
    e_var = est[1] / E - e_mean * e_mean
    a_e = (params['bn_edge_g'] * lax.rsqrt(e_var + 1e-5)).reshape(1, -1)
    c_e = (params['bn_edge_b'] - e_mean * a_e[0]).reshape(1, -1)

    xcur = xbn
    ef = e
    dims = [(16, 10), (32, 32), (32, 32)]
    for i, (nin, ein) in enumerate(dims):
        L = params['mp%d' % i]
        src, tgt = _gather2_call(xcur, row, col)
        aff = (a_e, c_e) if i == 0 else None
        ef, msg = _edge_call(nin, ein, src, tgt, ef, aff, L)
        agg2 = _scatter_call(msg, col, N)
        xcur = _update_call(agg2, xcur, L['root_w'], L['conv_b'])

    src, tgt = _gather2_call(xcur, row, col)
    return _pred_call(src, tgt, ef, params['pred'])


# speedup vs baseline: 3.7219x; 1.0092x over previous
import functools

import jax
import jax.numpy as jnp
from jax import lax
from jax.experimental import pallas as pl
from jax.experimental.pallas import tpu as pltpu
from jax.experimental.pallas import tpu_sc as plsc

LEAK = 0.1
EOUT = 32
NOUT = 32
EB = 4000


def _lk(v):
    return jnp.where(v >= 0, v, LEAK * v)


def _stats_body(x_ref, e_ref, g_ref, b_ref, xbn_ref, est_ref):
    @pl.when(pl.program_id(0) == 0)
    def _():
        x = x_ref[...]
        m = jnp.mean(x, axis=0, keepdims=True)
        v = jnp.mean((x - m) * (x - m), axis=0, keepdims=True)
        xbn_ref[...] = ((x - m) * lax.rsqrt(v + 1e-5) * g_ref[...]
                        + b_ref[...])
        est_ref[...] = jnp.zeros_like(est_ref)

    ch = e_ref[...]
    s = jnp.sum(ch, axis=0, keepdims=True)
    ss = jnp.sum(ch * ch, axis=0, keepdims=True)
    est_ref[...] += jnp.concatenate([s, ss], axis=0)


def _stats_call(x, e, g, b):
    N, D = x.shape
    E, ein = e.shape
    CH = 10000
    full = lambda a: pl.BlockSpec(a.shape, lambda i: (0,) * a.ndim)
    g2 = g.reshape(1, D)
    b2 = b.reshape(1, D)
    return pl.pallas_call(
        _stats_body,
        grid=(E // CH,),
        in_specs=[full(x), pl.BlockSpec((CH, ein), lambda i: (i, 0)),
                  full(g2), full(b2)],
        out_specs=(pl.BlockSpec((N, D), lambda i: (0, 0)),
                   pl.BlockSpec((2, ein), lambda i: (0, 0))),
        out_shape=(jax.ShapeDtypeStruct((N, D), jnp.float32),
                   jax.ShapeDtypeStruct((2, ein), jnp.float32)),
    )(x, e, g2, b2)


_NC, _NS = 2, 16
_NW = _NC * _NS
_CHK = 256


_GRP = 2
_GW = _GRP * _CHK


def _gather_body(EPW, D, table_ref, row_ref, col_ref, src_ref, tgt_ref,
                 ridx, cidx, bufa, bufb, sem0, sem1):
    wid = lax.axis_index("s") * _NC + lax.axis_index("c")
    base = wid * EPW
    pltpu.sync_copy(row_ref.at[pl.ds(base, EPW)], ridx)
    pltpu.sync_copy(col_ref.at[pl.ds(base, EPW)], cidx)
    ngrp = (EPW + _GW - 1) // _GW
    offs = [min(g * _GW, EPW - _GW) for g in range(ngrp)]
    sems = (sem0, sem1)
    pending = {}

    def issue(g):
        slot = g % 2
        cps = []
        for j in range(_GRP):
            lo = offs[g] + j * _CHK
            cps.append(pltpu.async_copy(
                table_ref.at[ridx.at[pl.ds(lo, _CHK)]],
                bufa.at[slot, pl.ds(j * _CHK, _CHK)], sems[slot]))
            cps.append(pltpu.async_copy(
                table_ref.at[cidx.at[pl.ds(lo, _CHK)]],
                bufb.at[slot, pl.ds(j * _CHK, _CHK)], sems[slot]))
        pending[g] = cps

    def drain(g):
        slot = g % 2
        for cp in pending.pop(g):
            cp.wait()
        pltpu.sync_copy(bufa.at[slot], src_ref.at[pl.ds(base + offs[g], _GW)])
        pltpu.sync_copy(bufb.at[slot], tgt_ref.at[pl.ds(base + offs[g], _GW)])

    issue(0)
    for g in range(1, ngrp):
        issue(g)
        drain(g - 1)
    drain(ngrp - 1)


def _gather2_call(table, row, col):
    N, D = table.shape
    E = row.shape[0]
    EPW = E // _NW
    mesh = plsc.VectorSubcoreMesh(core_axis_name="c", subcore_axis_name="s")
    kfn = pl.kernel(
        functools.partial(_gather_body, EPW, D),
        out_type=(
            jax.ShapeDtypeStruct((E, D), jnp.float32),
            jax.ShapeDtypeStruct((E, D), jnp.float32),
        ),
        mesh=mesh,
        compiler_params=pltpu.CompilerParams(use_tc_tiling_on_sc=False),
        scratch_types=[
            pltpu.VMEM((EPW,), jnp.int32),
            pltpu.VMEM((EPW,), jnp.int32),
            pltpu.VMEM((2, _GW, D), jnp.float32),
            pltpu.VMEM((2, _GW, D), jnp.float32),
            pltpu.SemaphoreType.DMA,
            pltpu.SemaphoreType.DMA,
        ],
    )
    return kfn(table, row, col)


def _scatter_body(N, NFULL, msg_ref, col_ref, zeros_ref, agg_ref,
                  idx2, buf2, shared, sem0, sem1):
    cid = lax.axis_index("c")
    sid = lax.axis_index("s")
    wid = sid * _NC + cid

    @pl.when(sid == 0)
    def _():
        pltpu.sync_copy(zeros_ref, shared)

    plsc.subcore_barrier()

    per_w = NFULL // _NW
    base = wid * per_w * _CHK
    sems = (sem0, sem1)

    def issue(k):
        slot = k % 2
        ci = pltpu.async_copy(col_ref.at[pl.ds(base + k * _CHK, _CHK)],
                              idx2.at[slot], sems[slot])
        cm = pltpu.async_copy(msg_ref.at[pl.ds(base + k * _CHK, _CHK)],
                              buf2.at[slot], sems[slot])
        return (ci, cm)

    pend = issue(0)
    for k in range(per_w):
        nxt = issue(k + 1) if k + 1 < per_w else None
        for cp in pend:
            cp.wait()
        slot = k % 2
        pltpu.sync_copy(buf2.at[slot], shared.at[idx2.at[slot]], add=True)
        pend = nxt

    tail = NFULL - per_w * _NW

    @pl.when(wid < tail)
    def _():
        off = per_w * _NW * _CHK + wid * _CHK
        pltpu.sync_copy(col_ref.at[pl.ds(off, _CHK)], idx2.at[0])
        pltpu.sync_copy(msg_ref.at[pl.ds(off, _CHK)], buf2.at[0])
        pltpu.sync_copy(buf2.at[0], shared.at[idx2.at[0]], add=True)

    plsc.subcore_barrier()
    rows = N // _NS
    pltpu.sync_copy(shared.at[pl.ds(sid * rows, rows)],
                    agg_ref.at[cid, pl.ds(sid * rows, rows)])


def _scatter_call(msg, col, N):
    E, D = msg.shape
    nfull = E // _CHK
    zeros = jnp.zeros((N, D), jnp.float32)
    mesh = plsc.VectorSubcoreMesh(core_axis_name="c", subcore_axis_name="s")
    kfn = pl.kernel(
        functools.partial(_scatter_body, N, nfull),
        out_type=jax.ShapeDtypeStruct((_NC, N, D), jnp.float32),
        mesh=mesh,
        compiler_params=pltpu.CompilerParams(use_tc_tiling_on_sc=False),
        scratch_types=[
            pltpu.VMEM((2, _CHK), jnp.int32),
            pltpu.VMEM((2, _CHK, D), jnp.float32),
            pltpu.VMEM_SHARED((N, D), jnp.float32),
            pltpu.SemaphoreType.DMA,
            pltpu.SemaphoreType.DMA,
        ],
    )
    return kfn(msg, col, zeros)


def _fold32(p):
    B, w = p.shape
    q = p[:, 0:min(128, w)]
    if q.shape[1] < 128:
        q = jnp.concatenate(
            [q, jnp.zeros((B, 128 - q.shape[1]), jnp.float32)], axis=1)
    c = 128
    while c + 128 <= w:
        q = q + p[:, c:c + 128]
        c += 128
    if c < w:
        r = w - c
        q = q + jnp.concatenate(
            [p[:, c:w], jnp.zeros((B, 128 - r), jnp.float32)], axis=1)
    h = q[:, 0:64] + q[:, 64:128]
    return h[:, 0:32] + h[:, 32:64]


def _edge_body(nin, ein, has_aff, *refs):
    if has_aff:
        (src_ref, tgt_ref, e_ref, a_ref, c_ref, r1_ref, r2_ref,
         c1_ref, c2_ref, beeb_ref,
         w1_ref, b1_ref, w2_ref, b2_ref, enew_ref, msg_ref) = refs
    else:
        (src_ref, tgt_ref, e_ref, r1_ref, r2_ref,
         c1_ref, c2_ref, beeb_ref,
         w1_ref, b1_ref, w2_ref, b2_ref, enew_ref, msg_ref) = refs
    dot = lambda a, b: jnp.dot(a, b, preferred_element_type=jnp.float32)
    s = src_ref[...]
    t = tgt_ref[...]
    ef = e_ref[...]
    if has_aff:
        ef = ef * a_ref[...] + c_ref[...]
    s_exp = dot(s, r1_ref[...])
    a1 = dot(t, c1_ref[...])
    z = _fold32(s_exp * a1)
    e_exp = dot(ef, r2_ref[...])
    a2 = dot(z, c2_ref[...])
    z2 = _fold32(e_exp * a2)
    en = _lk(z2 + beeb_ref[...])
    enew_ref[...] = en
    h1 = _lk(dot(en, w1_ref[...]) + b1_ref[...])
    h2 = _lk(dot(h1, w2_ref[...]) + b2_ref[...])
    msg_ref[...] = _fold32(s_exp * h2)


def _edge_call(nin, ein, src, tgt, ef, aff, L):
    E = src.shape[0]
    c1 = L['bst_w'].transpose(2, 1, 0).reshape(nin, nin * EOUT)
    c2 = L['bee_w'].transpose(1, 2, 0).reshape(EOUT, ein * EOUT)
    r1 = (jnp.arange(nin * EOUT) // EOUT
          == jnp.arange(nin)[:, None]).astype(jnp.float32)
    r2 = (jnp.arange(ein * EOUT) // EOUT
          == jnp.arange(ein)[:, None]).astype(jnp.float32)
    beeb = L['bee_b'].reshape(1, EOUT)
    w1 = L['nn1_w'].T
    b1 = L['nn1_b'].reshape(1, nin)
    w2 = L['nn2_w'].T
    b2 = L['nn2_b'].reshape(1, nin * NOUT)
    grid = (E // EB,)
    eb_spec = lambda d: pl.BlockSpec((EB, d), lambda g: (g, 0))
    w_spec = lambda a: pl.BlockSpec(a.shape, lambda g: (0,) * a.ndim)
    ops = [src, tgt, ef]
    specs = [eb_spec(nin), eb_spec(nin), eb_spec(ein)]
    if aff is not None:
        ops += [aff[0], aff[1]]
        specs += [w_spec(aff[0]), w_spec(aff[1])]
    ops += [r1, r2, c1, c2, beeb, w1, b1, w2, b2]
    specs += [w_spec(o) for o in (r1, r2, c1, c2, beeb, w1, b1, w2, b2)]
    return pl.pallas_call(
        functools.partial(_edge_body, nin, ein, aff is not None),
        grid=grid,
        in_specs=specs,
        out_specs=(eb_spec(EOUT), eb_spec(NOUT)),
        out_shape=(
            jax.ShapeDtypeStruct((E, EOUT), jnp.float32),
            jax.ShapeDtypeStruct((E, NOUT), jnp.float32),
        ),
    )(*ops)


def _update_body(agg_ref, x_ref, rw_ref, cb_ref, out_ref):
    out_ref[...] = (agg_ref[0] + agg_ref[1]
                    + jnp.dot(x_ref[...], rw_ref[...],
                              preferred_element_type=jnp.float32)
                    + cb_ref[...])


def _update_call(agg2, x, rw, cb):
    N = x.shape[0]
    return pl.pallas_call(
        _update_body,
        out_shape=jax.ShapeDtypeStruct((N, NOUT), jnp.float32),
    )(agg2, x, rw.T, cb.reshape(1, NOUT))


def _pred_body(s_ref, t_ref, e_ref, w0s_ref, w0t_ref, w0e_ref, b0_ref,
               w1_ref, b1_ref, w2_ref, b2_ref, w3_ref, b3_ref,
               w4_ref, b4_ref, out_ref):
    dot = lambda a, b: jnp.dot(a, b, preferred_element_type=jnp.float32)
    h = (dot(s_ref[...], w0s_ref[...]) + dot(t_ref[...], w0t_ref[...])
         + dot(e_ref[...], w0e_ref[...]) + b0_ref[...])
    h = _lk(h)
    h = _lk(dot(h, w1_ref[...]) + b1_ref[...])
    h = _lk(dot(h, w2_ref[...]) + b2_ref[...])
    h = _lk(dot(h, w3_ref[...]) + b3_ref[...])
    out_ref[...] = dot(h, w4_ref[...]) + b4_ref[...]


def _pred_call(src, tgt, ef, P):
    E = src.shape[0]
    w0 = P['w0']
    ops = [src, tgt, ef,
           w0[:, :32].T, w0[:, 32:64].T, w0[:, 64:96].T, P['b0'].reshape(1, -1),
           P['w1'].T, P['b1'].reshape(1, -1),
           P['w2'].T, P['b2'].reshape(1, -1),
           P['w3'].T, P['b3'].reshape(1, -1),
           P['w4'].T, P['b4'].reshape(1, -1)]
    eb_spec = lambda d: pl.BlockSpec((EB, d), lambda g: (g, 0))
    w_spec = lambda a: pl.BlockSpec(a.shape, lambda g: (0,) * a.ndim)
    specs = [eb_spec(32), eb_spec(32), eb_spec(32)]
    specs += [w_spec(o) for o in ops[3:]]
    return pl.pallas_call(
        _pred_body,
        grid=(E // EB,),
        in_specs=specs,
        out_specs=eb_spec(2),
        out_shape=jax.ShapeDtypeStruct((E, 2), jnp.float32),
    )(*ops)


def kernel(x, edge_index, e, xbatch, params):
    N = x.shape[0]
    E = e.shape[0]
    row = edge_index[0]
    col = edge_index[1]

    xbn, est = _stats_call(x, e, params['bn_node_g'], params['bn_node_b'])
    e_mean = est[0] / E
    e_var = est[1] / E - e_mean * e_mean
    a_e = (params['bn_edge_g'] * lax.rsqrt(e_var + 1e-5)).reshape(1, -1)
    c_e = (params['bn_edge_b'] - e_mean * a_e[0]).reshape(1, -1)

    xcur = xbn
    ef = e
    dims = [(16, 10), (32, 32), (32, 32)]
    for i, (nin, ein) in enumerate(dims):
        L = params['mp%d' % i]
        src, tgt = _gather2_call(xcur, row, col)
        aff = (a_e, c_e) if i == 0 else None
        ef, msg = _edge_call(nin, ein, src, tgt, ef, aff, L)
        agg2 = _scatter_call(msg, col, N)
        xcur = _update_call(agg2, xcur, L['root_w'], L['conv_b'])

    src, tgt = _gather2_call(xcur, row, col)
    return _pred_call(src, tgt, ef, params['pred'])
